# baseline (speedup 1.0000x reference)
import dataclasses
import functools

import jax
import jax.numpy as jnp
from jax import lax
from jax.experimental import pallas as pl
from jax.experimental.pallas import tpu as pltpu
from jax.experimental.pallas import tpu_sc as plsc

N = 10000
E = 320000
D = 128
H = 512
G = 64

NC = 2
NS = 16
NW = NC * NS
L = 16
B = 128

NB = -(-(E // NW) // B)
NB = -(-NB // 6) * 6
EPW = NB * B
E_PAD = EPW * NW

ROWS_PER_SUB = N // NS
F = 128


def _sc_edge_agg(num_chunks):
  mesh = plsc.VectorSubcoreMesh(core_axis_name="c", subcore_axis_name="s")

  def body(*refs):
    tables = refs[:num_chunks]
    epack_hbm, zeros_hbm, out_hbm = refs[num_chunks:num_chunks + 3]
    (e0, e1, e2, r0, r1, acc_sh, esem, gsem, ssem) = refs[num_chunks + 3:]
    ebufs = (e0, e1, e2)
    rbufs = (r0, r1)

    core = lax.axis_index("c")
    sub = lax.axis_index("s")
    wid = sub * NC + core

    def ecopy(b, k):
      return pltpu.make_async_copy(epack_hbm.at[wid].at[b], ebufs[k % 3],
                                   esem)

    def gather(c, b, k):
      return pltpu.make_async_copy(tables[c].at[ebufs[k % 3].at[0]],
                                   rbufs[k % 2], gsem)

    def scatter(b, k):
      return pltpu.make_async_copy(rbufs[k % 2],
                                   acc_sh.at[ebufs[k % 3].at[1]], ssem)

    def scale(k):
      ebuf = ebufs[k % 3]
      rbuf = rbufs[k % 2]

      @pl.loop(0, B, step=4)
      def _(i):
        for u in range(4):
          ii = i + u
          w16 = plsc.bitcast(
              plsc.load_gather(ebuf, [jnp.full((L,), 2, jnp.int32),
                                      jnp.full((L,), ii, jnp.int32)]),
              jnp.float32)
          for j in range(F // L):
            sl = (ii, pl.ds(j * L, L))
            rbuf[sl] = rbuf[sl] * w16

    for c in range(num_chunks):
      @pl.when(sub == 0)
      def _():
        pltpu.sync_copy(zeros_hbm, acc_sh)
      plsc.subcore_barrier()

      ecopy(0, 0).start()
      ecopy(1, 1).start()
      ecopy(0, 0).wait()
      gather(c, 0, 0).start()

      @pl.loop(0, NB, step=6)
      def _(b0):
        for k in range(6):
          b = b0 + k
          gather(c, b, k).wait()

          @pl.when(b >= 1)
          def _():
            scatter(b - 1, k - 1).wait()

          @pl.when(b + 2 < NB)
          def _():
            ecopy(b + 2, k + 2).start()

          @pl.when(b + 1 < NB)
          def _():
            ecopy(b + 1, k + 1).wait()
            gather(c, b + 1, k + 1).start()

          scale(k)
          scatter(b, k).start(add=True)

      scatter(NB - 1, NB - 1).wait()

      plsc.subcore_barrier()
      @pl.when(sub == 0)
      def _():
        pltpu.sync_copy(acc_sh, out_hbm.at[c].at[core])
      plsc.subcore_barrier()

  cp = pltpu.CompilerParams()
  if "needs_layout_passes" in pltpu.CompilerParams.__dataclass_fields__:
    cp = dataclasses.replace(cp, needs_layout_passes=False)
  kern = pl.kernel(
      body,
      mesh=mesh,
      compiler_params=cp,
      out_type=jax.ShapeDtypeStruct((num_chunks, NC, N, F), jnp.float32),
      scratch_types=[
          pltpu.VMEM((3, B), jnp.int32),
          pltpu.VMEM((3, B), jnp.int32),
          pltpu.VMEM((3, B), jnp.int32),
          pltpu.VMEM((B, F), jnp.float32),
          pltpu.VMEM((B, F), jnp.float32),
          pltpu.VMEM_SHARED((N, F), jnp.float32),
          pltpu.SemaphoreType.DMA,
          pltpu.SemaphoreType.DMA,
          pltpu.SemaphoreType.DMA,
      ],
  )
  return kern


_sc_agg1 = _sc_edge_agg(1)
_sc_agg4 = _sc_edge_agg(4)


def _tc1_body(agg_ref, x_ref, wrel_ref, b_ref, wroot_ref, *out_refs):
  agg = agg_ref[0] + agg_ref[1]
  h = lax.dot(agg, wrel_ref[...], precision=lax.Precision.HIGHEST,
              preferred_element_type=jnp.float32)
  h += lax.dot(x_ref[...], wroot_ref[...], precision=lax.Precision.HIGHEST,
               preferred_element_type=jnp.float32)
  h = jnp.maximum(h + b_ref[...], 0.0)
  for c in range(4):
    out_refs[c][...] = h[:, c * F:(c + 1) * F]


def _tc2_body(agg_ref, h1c0, h1c1, h1c2, h1c3, batch_ref, wrel_ref, b_ref,
              wroot_ref, wl1_ref, bl1_ref, wl2_ref, bl2_ref, wl3_ref, bl3_ref,
              out_ref, pool_acc, cnt_acc):
  h1_refs = (h1c0, h1c1, h1c2, h1c3)
  i = pl.program_id(0)
  nsteps = pl.num_programs(0)

  @pl.when(i == 0)
  def _():
    pool_acc[...] = jnp.zeros_like(pool_acc)
    cnt_acc[...] = jnp.zeros_like(cnt_acc)

  h2 = jnp.zeros((BLK, H), jnp.float32) + b_ref[...]
  for c in range(4):
    a = agg_ref[2 * c] + agg_ref[2 * c + 1]
    h2 += lax.dot(a, wrel_ref[pl.ds(c * F, F), :],
                  precision=lax.Precision.HIGHEST,
                  preferred_element_type=jnp.float32)
    h2 += lax.dot(h1_refs[c][...], wroot_ref[pl.ds(c * F, F), :],
                  precision=lax.Precision.HIGHEST,
                  preferred_element_type=jnp.float32)
  h2 = jnp.maximum(h2, 0.0)

  gids = lax.broadcasted_iota(jnp.int32, (G, BLK), 0)
  bids = batch_ref[0, :, :]
  onehot = (gids == bids).astype(jnp.float32)
  pool_acc[...] += lax.dot(onehot, h2, precision=lax.Precision.HIGHEST,
                           preferred_element_type=jnp.float32)
  cnt_acc[...] += jnp.sum(onehot, axis=1, keepdims=True)

  @pl.when(i == nsteps - 1)
  def _():
    pooled = pool_acc[...] / jnp.maximum(cnt_acc[...], 1.0)
    m = jnp.maximum(lax.dot(pooled, wl1_ref[...],
                            precision=lax.Precision.HIGHEST,
                            preferred_element_type=jnp.float32)
                    + bl1_ref[...], 0.0)
    m = jnp.maximum(lax.dot(m, wl2_ref[...],
                            precision=lax.Precision.HIGHEST,
                            preferred_element_type=jnp.float32)
                    + bl2_ref[...], 0.0)
    out_ref[...] = (jnp.sum(m * wl3_ref[...], axis=1, keepdims=True)
                    + bl3_ref[...])


BLK = 1000


def kernel(x, edge_index, edge_attr, batch,
           W1_rel, b1_rel, W1_root, W2_rel, b2_rel, W2_root,
           Wl1, bl1, Wl2, bl2, Wl3, bl3):
  src = edge_index[0]
  dst = edge_index[1]
  pad = E_PAD - E
  src_p = jnp.concatenate([src, jnp.zeros((pad,), jnp.int32)]).reshape(
      NW, NB, B)
  dst_p = jnp.concatenate([dst, jnp.zeros((pad,), jnp.int32)]).reshape(
      NW, NB, B)
  w_bits = lax.bitcast_convert_type(
      jnp.concatenate([edge_attr, jnp.zeros((pad,), jnp.float32)]),
      jnp.int32).reshape(NW, NB, B)
  epack = jnp.stack([src_p, dst_p, w_bits], axis=2)

  zeros_nf = jnp.zeros((N, F), jnp.float32)

  agg1 = _sc_agg1(x, epack, zeros_nf)

  ngrid = N // BLK
  h1c = pl.pallas_call(
      _tc1_body,
      grid=(ngrid,),
      in_specs=[
          pl.BlockSpec((2, BLK, D), lambda i: (0, i, 0)),
          pl.BlockSpec((BLK, D), lambda i: (i, 0)),
          pl.BlockSpec((D, H), lambda i: (0, 0)),
          pl.BlockSpec((1, H), lambda i: (0, 0)),
          pl.BlockSpec((D, H), lambda i: (0, 0)),
      ],
      out_specs=[pl.BlockSpec((BLK, F), lambda i: (i, 0))] * 4,
      out_shape=[jax.ShapeDtypeStruct((N, F), jnp.float32)] * 4,
  )(agg1[0], x, W1_rel.T, b1_rel.reshape(1, H), W1_root.T)

  agg2 = _sc_agg4(h1c[0], h1c[1], h1c[2], h1c[3], epack, zeros_nf)
  agg2 = agg2.reshape(8, N, F)

  out = pl.pallas_call(
      _tc2_body,
      grid=(ngrid,),
      in_specs=[
          pl.BlockSpec((8, BLK, F), lambda i: (0, i, 0)),
          pl.BlockSpec((BLK, F), lambda i: (i, 0)),
          pl.BlockSpec((BLK, F), lambda i: (i, 0)),
          pl.BlockSpec((BLK, F), lambda i: (i, 0)),
          pl.BlockSpec((BLK, F), lambda i: (i, 0)),
          pl.BlockSpec((1, 1, BLK), lambda i: (i, 0, 0)),
          pl.BlockSpec((H, H), lambda i: (0, 0)),
          pl.BlockSpec((1, H), lambda i: (0, 0)),
          pl.BlockSpec((H, H), lambda i: (0, 0)),
          pl.BlockSpec((H, G), lambda i: (0, 0)),
          pl.BlockSpec((1, G), lambda i: (0, 0)),
          pl.BlockSpec((G, 16), lambda i: (0, 0)),
          pl.BlockSpec((1, 16), lambda i: (0, 0)),
          pl.BlockSpec((1, 16), lambda i: (0, 0)),
          pl.BlockSpec((1, 1), lambda i: (0, 0)),
      ],
      out_specs=pl.BlockSpec((G, 1), lambda i: (0, 0)),
      out_shape=jax.ShapeDtypeStruct((G, 1), jnp.float32),
      scratch_shapes=[
          pltpu.VMEM((G, H), jnp.float32),
          pltpu.VMEM((G, 1), jnp.float32),
      ],
  )(agg2,
    h1c[0], h1c[1], h1c[2], h1c[3],
    batch.reshape(ngrid, 1, BLK),
    W2_rel.T, b2_rel.reshape(1, H), W2_root.T,
    Wl1.T, bl1.reshape(1, G), Wl2.T, bl2.reshape(1, 16),
    Wl3, bl3.reshape(1, 1))
  return out


# speedup vs baseline: 1.6235x; 1.6235x over previous
import dataclasses
import functools

import jax
import jax.numpy as jnp
from jax import lax
from jax.experimental import pallas as pl
from jax.experimental.pallas import tpu as pltpu
from jax.experimental.pallas import tpu_sc as plsc

N = 10000
E = 320000
D = 128
H = 512
G = 64

NC = 2
NS = 16
NW = NC * NS
L = 16
B = 128

NB = -(-(E // NW) // B)
NB = -(-NB // 16) * 16
SUP = NB // 8
EPW = NB * B
E_PAD = EPW * NW

ROWS_PER_SUB = N // NS
F = 128


def _sc_edge_agg(num_chunks):
  mesh = plsc.VectorSubcoreMesh(core_axis_name="c", subcore_axis_name="s")

  def body(table_hbm, eidx_hbm, ewts_hbm, zeros_hbm, out_hbm,
           e0, e1, w0, w1, r0, r1, acc_sh, esem, gsem, ssem):
    ebufs = (e0, e1)
    wbufs = (w0, w1)
    rbufs = (r0, r1)

    core = lax.axis_index("c")
    sub = lax.axis_index("s")
    wid = sub * NC + core

    def ecopies(c, s, sk):
      return (pltpu.make_async_copy(eidx_hbm.at[c].at[wid].at[s],
                                    ebufs[sk % 2], esem),
              pltpu.make_async_copy(ewts_hbm.at[wid].at[s], wbufs[sk % 2],
                                    esem))

    def gather(k):
      return pltpu.make_async_copy(
          table_hbm.at[ebufs[(k // 8) % 2].at[k % 8].at[0]],
          rbufs[k % 2], gsem)

    def scatter(k):
      return pltpu.make_async_copy(
          rbufs[k % 2],
          acc_sh.at[ebufs[(k // 8) % 2].at[k % 8].at[1]], ssem)

    def scale(k):
      wbuf = wbufs[(k // 8) % 2]
      j = k % 8
      rbuf = rbufs[k % 2]

      @pl.loop(0, B, step=4)
      def _(i):
        for u in range(4):
          ii = i + u
          w16 = plsc.load_gather(
              wbuf, [jnp.full((L,), j * B + ii, jnp.int32)])
          for jj in range(F // L):
            sl = (ii, pl.ds(jj * L, L))
            rbuf[sl] = rbuf[sl] * w16

    @pl.loop(0, num_chunks)
    def _(c):
      @pl.when(sub == 0)
      def _():
        pltpu.sync_copy(zeros_hbm, acc_sh)
      plsc.subcore_barrier()

      for d in ecopies(c, 0, 0):
        d.start()
      for d in ecopies(c, 0, 0):
        d.wait()
      gather(0).start()

      @pl.loop(0, NB, step=16)
      def _(b0):
        for k in range(16):
          b = b0 + k
          gather(k).wait()

          @pl.when(b >= 1)
          def _():
            scatter(k - 1).wait()

          if k % 8 == 0:
            @pl.when(b + 8 < NB)
            def _():
              for d in ecopies(c, b // 8 + 1, k // 8 + 1):
                d.start()

          @pl.when(b + 1 < NB)
          def _():
            if (k + 1) % 8 == 0:
              for d in ecopies(c, 0, k // 8 + 1):
                d.wait()
            gather(k + 1).start()

          scale(k)
          scatter(k).start(add=True)

      scatter(NB - 1).wait()

      plsc.subcore_barrier()
      @pl.when(sub == 0)
      def _():
        pltpu.sync_copy(acc_sh, out_hbm.at[c].at[core])
      plsc.subcore_barrier()

  cp = pltpu.CompilerParams()
  if "needs_layout_passes" in pltpu.CompilerParams.__dataclass_fields__:
    cp = dataclasses.replace(cp, needs_layout_passes=False)
  kern = pl.kernel(
      body,
      mesh=mesh,
      compiler_params=cp,
      out_type=jax.ShapeDtypeStruct((num_chunks, NC, N, F), jnp.float32),
      scratch_types=[
          pltpu.VMEM((8, 2, B), jnp.int32),
          pltpu.VMEM((8, 2, B), jnp.int32),
          pltpu.VMEM((8 * B,), jnp.float32),
          pltpu.VMEM((8 * B,), jnp.float32),
          pltpu.VMEM((B, F), jnp.float32),
          pltpu.VMEM((B, F), jnp.float32),
          pltpu.VMEM_SHARED((N, F), jnp.float32),
          pltpu.SemaphoreType.DMA,
          pltpu.SemaphoreType.DMA,
          pltpu.SemaphoreType.DMA,
      ],
  )
  return kern


_sc_agg1 = _sc_edge_agg(1)
_sc_agg4 = _sc_edge_agg(4)


def _bdot(a, b):
  return lax.dot(a.astype(jnp.bfloat16), b.astype(jnp.bfloat16),
                 preferred_element_type=jnp.float32)


def _tc1_body(agg_ref, x_ref, wrel_ref, b_ref, wroot_ref, out_ref):
  agg = agg_ref[0] + agg_ref[1]
  h = _bdot(agg, wrel_ref[...]) + b_ref[...]
  h += _bdot(x_ref[...], wroot_ref[...])
  h = jnp.maximum(h, 0.0)
  for c in range(4):
    out_ref[c] = h[:, c * F:(c + 1) * F]


def _tc2_body(agg_ref, h1_ref, batch_ref, wrel_ref, b_ref,
              wroot_ref, wl1_ref, bl1_ref, wl2_ref, bl2_ref, wl3_ref, bl3_ref,
              out_ref, pool_acc, cnt_acc):
  i = pl.program_id(0)
  nsteps = pl.num_programs(0)

  @pl.when(i == 0)
  def _():
    pool_acc[...] = jnp.zeros_like(pool_acc)
    cnt_acc[...] = jnp.zeros_like(cnt_acc)

  h2 = jnp.zeros((BLK, H), jnp.float32) + b_ref[...]
  for c in range(4):
    a = agg_ref[2 * c] + agg_ref[2 * c + 1]
    h2 += _bdot(a, wrel_ref[pl.ds(c * F, F), :])
    h2 += _bdot(h1_ref[c], wroot_ref[pl.ds(c * F, F), :])
  h2 = jnp.maximum(h2, 0.0)

  gids = lax.broadcasted_iota(jnp.int32, (G, BLK), 0)
  bids = batch_ref[0, :, :]
  onehot = (gids == bids).astype(jnp.float32)
  pool_acc[...] += lax.dot(onehot, h2, precision=lax.Precision.HIGHEST,
                           preferred_element_type=jnp.float32)
  cnt_acc[...] += jnp.sum(onehot, axis=1, keepdims=True)

  @pl.when(i == nsteps - 1)
  def _():
    pooled = pool_acc[...] / jnp.maximum(cnt_acc[...], 1.0)
    m = jnp.maximum(_bdot(pooled, wl1_ref[...]) + bl1_ref[...], 0.0)
    m = jnp.maximum(_bdot(m, wl2_ref[...]) + bl2_ref[...], 0.0)
    out_ref[...] = _bdot(m, wl3_ref[...])[:, 0:1] + bl3_ref[...]


BLK = 1000


def kernel(x, edge_index, edge_attr, batch,
           W1_rel, b1_rel, W1_root, W2_rel, b2_rel, W2_root,
           Wl1, bl1, Wl2, bl2, Wl3, bl3):
  perm = jnp.argsort(edge_index[1], stable=True)
  src = edge_index[0][perm]
  dst = edge_index[1][perm]
  edge_attr = edge_attr[perm]
  pad = E_PAD - E
  src_p = jnp.concatenate([src, jnp.zeros((pad,), jnp.int32)]).reshape(
      NW, NB, B)
  dst_p = jnp.concatenate([dst, jnp.zeros((pad,), jnp.int32)]).reshape(
      NW, NB, B)
  def make_eidx(c_chunks):
    offs = jnp.arange(c_chunks, dtype=jnp.int32) * N
    srcs = src_p[None] + offs[:, None, None, None]
    dsts = jnp.broadcast_to(dst_p[None], srcs.shape)
    return jnp.stack([srcs, dsts], axis=3).reshape(
        c_chunks, NW, SUP, 8, 2, B)

  eidx1 = make_eidx(1)
  eidx4 = make_eidx(4)
  ewts = jnp.concatenate([edge_attr, jnp.zeros((pad,), jnp.float32)]
                         ).reshape(NW, SUP, 8 * B)

  zeros_nf = jnp.zeros((N, F), jnp.float32)

  agg1 = _sc_agg1(x, eidx1, ewts, zeros_nf)

  ngrid = N // BLK
  h1c = pl.pallas_call(
      _tc1_body,
      grid=(ngrid,),
      in_specs=[
          pl.BlockSpec((2, BLK, D), lambda i: (0, i, 0)),
          pl.BlockSpec((BLK, D), lambda i: (i, 0)),
          pl.BlockSpec((D, H), lambda i: (0, 0)),
          pl.BlockSpec((1, H), lambda i: (0, 0)),
          pl.BlockSpec((D, H), lambda i: (0, 0)),
      ],
      out_specs=pl.BlockSpec((4, BLK, F), lambda i: (0, i, 0)),
      out_shape=jax.ShapeDtypeStruct((4, N, F), jnp.float32),
  )(agg1[0], x, W1_rel.T, b1_rel.reshape(1, H), W1_root.T)

  agg2 = _sc_agg4(h1c.reshape(4 * N, F), eidx4, ewts, zeros_nf)
  agg2 = agg2.reshape(8, N, F)

  out = pl.pallas_call(
      _tc2_body,
      grid=(ngrid,),
      in_specs=[
          pl.BlockSpec((8, BLK, F), lambda i: (0, i, 0)),
          pl.BlockSpec((4, BLK, F), lambda i: (0, i, 0)),
          pl.BlockSpec((1, 1, BLK), lambda i: (i, 0, 0)),
          pl.BlockSpec((H, H), lambda i: (0, 0)),
          pl.BlockSpec((1, H), lambda i: (0, 0)),
          pl.BlockSpec((H, H), lambda i: (0, 0)),
          pl.BlockSpec((H, G), lambda i: (0, 0)),
          pl.BlockSpec((1, G), lambda i: (0, 0)),
          pl.BlockSpec((G, 16), lambda i: (0, 0)),
          pl.BlockSpec((1, 16), lambda i: (0, 0)),
          pl.BlockSpec((16, 128), lambda i: (0, 0)),
          pl.BlockSpec((1, 1), lambda i: (0, 0)),
      ],
      out_specs=pl.BlockSpec((G, 1), lambda i: (0, 0)),
      out_shape=jax.ShapeDtypeStruct((G, 1), jnp.float32),
      scratch_shapes=[
          pltpu.VMEM((G, H), jnp.float32),
          pltpu.VMEM((G, 1), jnp.float32),
      ],
  )(agg2,
    h1c,
    batch.reshape(ngrid, 1, BLK),
    W2_rel.T, b2_rel.reshape(1, H), W2_root.T,
    Wl1.T, bl1.reshape(1, G), Wl2.T, bl2.reshape(1, 16),
    jnp.pad(Wl3.T, ((0, 0), (0, 127))), bl3.reshape(1, 1))
  return out
